# baseline (speedup 1.0000x reference)
import functools

import jax
import jax.numpy as jnp
from jax import lax
from jax.experimental import pallas as pl
from jax.experimental.pallas import tpu as pltpu
from jax.experimental.pallas import tpu_sc as plsc

N_NODES = 10000
D_FEAT = 128
HID = 16
EMB = 100
ACT = 64

NC, NS, L = 2, 16, 16
NW = NC * NS
NP = 10240
ROWS_PT = NP // NS
CHUNK = 128
N_EDGES = 320000
E_TOT = N_EDGES + N_NODES
CPT = -(-E_TOT // (NW * CHUNK))
CPT += CPT % 2
EPT = CPT * CHUNK
EPAD = EPT * NW
DCPT = CPT * NC

_mesh = plsc.VectorSubcoreMesh(core_axis_name="c", subcore_axis_name="s")
_sc_params = pltpu.CompilerParams(use_tc_tiling_on_sc=False,
                                  needs_layout_passes=False)


def _rsqrt_newton(d):
    y = plsc.bitcast(jnp.int32(0x5F3759DF) - (plsc.bitcast(d, jnp.int32) >> 1),
                     jnp.float32)
    for _ in range(3):
        y = y * (1.5 - 0.5 * d * y * y)
    return y


@functools.partial(
    pl.kernel,
    out_type=jax.ShapeDtypeStruct((EPAD // CHUNK, CHUNK), jnp.float32),
    mesh=_mesh,
    compiler_params=_sc_params,
    scratch_types=[
        pltpu.VMEM((DCPT, CHUNK), jnp.int32),
        pltpu.VMEM((DCPT, CHUNK), jnp.float32),
        pltpu.VMEM((CPT, CHUNK), jnp.int32),
        pltpu.VMEM((CPT, CHUNK), jnp.float32),
        pltpu.VMEM((NP,), jnp.float32),
        pltpu.VMEM((ROWS_PT,), jnp.float32),
        pltpu.VMEM_SHARED((NP,), jnp.float32),
        pltpu.VMEM_SHARED((NP,), jnp.float32),
        pltpu.SemaphoreType.DMA,
        pltpu.SemaphoreType.DMA,
        pltpu.SemaphoreType.DMA,
        pltpu.SemaphoreType.DMA,
    ],
)
def _sc_norm(row_hbm, col_hbm, ew_hbm, norm_hbm,
             c_buf, w_buf, r_buf, n_buf, dis_buf, d_buf, acc, dis_sh,
             dsem0, dsem1, dsem2, dsem3):
    cid = lax.axis_index("c")
    sid = lax.axis_index("s")
    wid = sid * NC + cid

    def zb(i, carry):
        d_buf[pl.ds(i * L, L)] = jnp.zeros((L,), jnp.float32)
        return carry

    lax.fori_loop(0, ROWS_PT // L, zb, 0)
    pltpu.sync_copy(d_buf, acc.at[pl.ds(sid * ROWS_PT, ROWS_PT)])
    pltpu.sync_copy(col_hbm.at[pl.ds(sid * DCPT, DCPT), :], c_buf)
    pltpu.sync_copy(ew_hbm.at[pl.ds(sid * DCPT, DCPT), :], w_buf)
    plsc.subcore_barrier()

    dsems = [dsem0, dsem1, dsem2, dsem3]
    for t in range(4):
        pltpu.async_copy(w_buf.at[t], acc.at[c_buf.at[t]], dsems[t], add=True)

    def deg_body(i, carry):
        j = 4 * i
        for t in range(4):
            pltpu.make_async_copy(w_buf.at[j + t], acc.at[c_buf.at[j + t]],
                                  dsems[t]).wait()

            @pl.when(j + t + 4 < DCPT)
            def _():
                pltpu.async_copy(w_buf.at[j + t + 4],
                                 acc.at[c_buf.at[j + t + 4]],
                                 dsems[t], add=True)

        return carry

    lax.fori_loop(0, DCPT // 4, deg_body, 0)
    plsc.subcore_barrier()

    pltpu.sync_copy(acc.at[pl.ds(sid * ROWS_PT, ROWS_PT)], d_buf)
    for i in range(ROWS_PT // L):
        sl = pl.ds(i * L, L)
        d_buf[sl] = _rsqrt_newton(d_buf[sl])
    pltpu.sync_copy(d_buf, dis_sh.at[pl.ds(sid * ROWS_PT, ROWS_PT)])
    plsc.subcore_barrier()
    pltpu.sync_copy(dis_sh, dis_buf)

    pltpu.sync_copy(row_hbm.at[pl.ds(wid * CPT, CPT), :], r_buf)
    pltpu.sync_copy(col_hbm.at[pl.ds(wid * CPT, CPT), :],
                    c_buf.at[pl.ds(0, CPT), :])
    pltpu.sync_copy(ew_hbm.at[pl.ds(wid * CPT, CPT), :], n_buf)

    def norm_body(j, carry):
        for b in range(CHUNK // L):
            sl = pl.ds(b * L, L)
            dr = plsc.load_gather(dis_buf, [r_buf[j, sl]])
            dc = plsc.load_gather(dis_buf, [c_buf[j, sl]])
            n_buf[j, sl] = dr * n_buf[j, sl] * dc
        return carry

    lax.fori_loop(0, CPT, norm_body, 0)
    pltpu.sync_copy(n_buf, norm_hbm.at[pl.ds(wid * CPT, CPT), :])


_AGG_SCRATCH = [
    pltpu.VMEM((CPT, CHUNK), jnp.int32),
    pltpu.VMEM((CPT, CHUNK), jnp.int32),
    pltpu.VMEM((CPT, CHUNK), jnp.float32),
    pltpu.VMEM((CHUNK, HID), jnp.float32),
    pltpu.VMEM((CHUNK, HID), jnp.float32),
    pltpu.VMEM((CHUNK, HID), jnp.float32),
    pltpu.VMEM((CHUNK, HID), jnp.float32),
    pltpu.VMEM((ROWS_PT, HID), jnp.float32),
    pltpu.VMEM_SHARED((NP, HID), jnp.float32),
    pltpu.VMEM_SHARED((NP, HID), jnp.float32),
    pltpu.SemaphoreType.DMA,
    pltpu.SemaphoreType.DMA,
    pltpu.SemaphoreType.DMA,
    pltpu.SemaphoreType.DMA,
]


def _agg_main(row_hbm, col_hbm, norm_hbm, m_hbm,
              r_buf, c_buf, n_buf, g0, g1, s0, s1, o_buf, m_sh, acc,
              gsem0, gsem1, ssem0, ssem1, cid, sid, wid):
    def zb(i, carry):
        o_buf[i, :] = jnp.zeros((L,), jnp.float32)
        return carry

    lax.fori_loop(0, ROWS_PT, zb, 0)
    pltpu.sync_copy(o_buf, acc.at[pl.ds(sid * ROWS_PT, ROWS_PT), :])
    pltpu.sync_copy(row_hbm.at[pl.ds(wid * CPT, CPT), :], r_buf)
    pltpu.sync_copy(col_hbm.at[pl.ds(wid * CPT, CPT), :], c_buf)
    pltpu.sync_copy(norm_hbm.at[pl.ds(wid * CPT, CPT), :], n_buf)
    plsc.subcore_barrier()

    iota = lax.iota(jnp.int32, L)
    evecs = [iota + b * L for b in range(CHUNK // L)]
    fvecs = [jnp.full((L,), f, jnp.int32) for f in range(HID)]

    def scale(j, g_buf, s_buf):
        for b in range(CHUNK // L):
            nv = n_buf[j, pl.ds(b * L, L)]
            for f in range(HID):
                vals = plsc.load_gather(g_buf, [evecs[b], fvecs[f]])
                plsc.store_scatter(s_buf, [evecs[b], fvecs[f]], vals * nv)

    pltpu.async_copy(m_hbm.at[r_buf.at[0]], g0, gsem0)
    pltpu.async_copy(m_hbm.at[r_buf.at[1]], g1, gsem1)

    def body(i, carry):
        j = 2 * i
        pltpu.make_async_copy(m_hbm.at[r_buf.at[j]], g0, gsem0).wait()

        @pl.when(i > 0)
        def _():
            pltpu.make_async_copy(s0, acc.at[c_buf.at[j]], ssem0).wait()

        scale(j, g0, s0)
        pltpu.async_copy(s0, acc.at[c_buf.at[j]], ssem0, add=True)

        @pl.when(j + 2 < CPT)
        def _():
            pltpu.async_copy(m_hbm.at[r_buf.at[j + 2]], g0, gsem0)

        pltpu.make_async_copy(m_hbm.at[r_buf.at[j + 1]], g1, gsem1).wait()

        @pl.when(i > 0)
        def _():
            pltpu.make_async_copy(s1, acc.at[c_buf.at[j + 1]], ssem1).wait()

        scale(j + 1, g1, s1)
        pltpu.async_copy(s1, acc.at[c_buf.at[j + 1]], ssem1, add=True)

        @pl.when(j + 3 < CPT)
        def _():
            pltpu.async_copy(m_hbm.at[r_buf.at[j + 3]], g1, gsem1)

        return carry

    lax.fori_loop(0, CPT // 2, body, 0)
    pltpu.make_async_copy(s0, acc.at[c_buf.at[CPT - 2]], ssem0).wait()
    pltpu.make_async_copy(s1, acc.at[c_buf.at[CPT - 1]], ssem1).wait()
    plsc.subcore_barrier()


@functools.partial(
    pl.kernel,
    out_type=jax.ShapeDtypeStruct((NC, NP, HID), jnp.float32),
    mesh=_mesh,
    compiler_params=_sc_params,
    scratch_types=_AGG_SCRATCH,
)
def _sc_agg(row_hbm, col_hbm, norm_hbm, m_hbm, out_hbm,
            r_buf, c_buf, n_buf, g0, g1, s0, s1, o_buf, m_sh, acc,
            gsem0, gsem1, ssem0, ssem1):
    cid = lax.axis_index("c")
    sid = lax.axis_index("s")
    wid = sid * NC + cid
    _agg_main(row_hbm, col_hbm, norm_hbm, m_hbm,
              r_buf, c_buf, n_buf, g0, g1, s0, s1, o_buf, m_sh, acc,
              gsem0, gsem1, ssem0, ssem1, cid, sid, wid)
    pltpu.sync_copy(acc.at[pl.ds(sid * ROWS_PT, ROWS_PT), :], o_buf)
    pltpu.sync_copy(o_buf, out_hbm.at[cid, pl.ds(sid * ROWS_PT, ROWS_PT), :])


@functools.partial(
    pl.kernel,
    out_type=jax.ShapeDtypeStruct((NC, ACT, HID), jnp.float32),
    mesh=_mesh,
    compiler_params=_sc_params,
    scratch_types=_AGG_SCRATCH + [
        pltpu.VMEM((ACT,), jnp.int32),
        pltpu.VMEM((ACT, HID), jnp.float32),
    ],
)
def _sc_agg_gather(row_hbm, col_hbm, norm_hbm, m_hbm, pos_hbm, out_hbm,
                   r_buf, c_buf, n_buf, g0, g1, s0, s1, o_buf, m_sh, acc,
                   gsem0, gsem1, ssem0, ssem1, i_buf, ga_buf):
    cid = lax.axis_index("c")
    sid = lax.axis_index("s")
    wid = sid * NC + cid
    _agg_main(row_hbm, col_hbm, norm_hbm, m_hbm,
              r_buf, c_buf, n_buf, g0, g1, s0, s1, o_buf, m_sh, acc,
              gsem0, gsem1, ssem0, ssem1, cid, sid, wid)

    @pl.when(sid == 0)
    def _():
        pltpu.sync_copy(pos_hbm, i_buf)
        pltpu.async_copy(acc.at[i_buf], ga_buf, gsem0).wait()
        pltpu.sync_copy(ga_buf, out_hbm.at[cid])


def _tc_matmul(x, W):
    def body(x_ref, w_ref, o_ref):
        o_ref[...] = jnp.dot(x_ref[...], w_ref[...],
                             preferred_element_type=jnp.float32)

    return pl.pallas_call(
        body,
        out_shape=jax.ShapeDtypeStruct((x.shape[0], W.shape[1]), jnp.float32),
    )(x, W)


def _tc_comb_mm(agg, b, W):
    def body(a_ref, b_ref, w_ref, o_ref):
        h = jnp.maximum(a_ref[0] + a_ref[1] + b_ref[...], 0.0)
        o_ref[...] = jnp.dot(h, w_ref[...], preferred_element_type=jnp.float32)

    return pl.pallas_call(
        body,
        out_shape=jax.ShapeDtypeStruct((NP, W.shape[1]), jnp.float32),
    )(agg, b.reshape(1, -1), W)


def _tc_comb(agg, b):
    def body(a_ref, b_ref, o_ref):
        o_ref[...] = jnp.maximum(a_ref[0] + a_ref[1] + b_ref[...], 0.0)

    return pl.pallas_call(
        body,
        out_shape=jax.ShapeDtypeStruct((NP, HID), jnp.float32),
    )(agg, b.reshape(1, -1))


def _tc_emb(g2, pos2d, W3, b3):
    def body(g_ref, p_ref, w_ref, b_ref, o_ref):
        emb = jnp.dot(g_ref[0] + g_ref[1], w_ref[...],
                      preferred_element_type=jnp.float32) + b_ref[...]
        o_ref[...] = jnp.where(p_ref[...] == -1, jnp.float32(-1.0), emb)

    return pl.pallas_call(
        body,
        out_shape=jax.ShapeDtypeStruct((ACT, EMB), jnp.float32),
    )(g2, pos2d, W3, b3.reshape(1, -1))


def _tc_mlp(flat, Wf1, bf1, Wf2, bf2, Wf3, bf3):
    def body(f_ref, w1_ref, b1_ref, w2_ref, b2_ref, w3_ref, b3_ref, o_ref):
        z = jnp.maximum(jnp.dot(f_ref[...], w1_ref[...],
                                preferred_element_type=jnp.float32)
                        + b1_ref[...], 0.0)
        z = jnp.maximum(jnp.dot(z, w2_ref[...],
                                preferred_element_type=jnp.float32)
                        + b2_ref[...], 0.0)
        o_ref[...] = jnp.dot(z, w3_ref[...],
                             preferred_element_type=jnp.float32) + b3_ref[...]

    return pl.pallas_call(
        body,
        out_shape=jax.ShapeDtypeStruct((1, ACT), jnp.float32),
    )(flat, Wf1, bf1.reshape(1, -1), Wf2, bf2.reshape(1, -1),
      Wf3, bf3.reshape(1, -1))


def kernel(x, edge_index, edge_weight, pos, W1, b1, W2, b2, W3, b3,
           Wf1, bf1, Wf2, bf2, Wf3, bf3):
    ei = edge_index.astype(jnp.int32)
    loop = jnp.arange(N_NODES, dtype=jnp.int32)
    pad = EPAD - E_TOT
    row = jnp.concatenate([ei[0], loop, jnp.zeros((pad,), jnp.int32)])
    col = jnp.concatenate([ei[1], loop, jnp.zeros((pad,), jnp.int32)])
    ew = jnp.concatenate([edge_weight.astype(jnp.float32),
                          jnp.ones((N_NODES,), jnp.float32),
                          jnp.zeros((pad,), jnp.float32)])
    row2d = row.reshape(EPAD // CHUNK, CHUNK)
    col2d = col.reshape(EPAD // CHUNK, CHUNK)
    ew2d = ew.reshape(EPAD // CHUNK, CHUNK)

    x_pad = jnp.concatenate(
        [x.astype(jnp.float32), jnp.zeros((NP - N_NODES, D_FEAT), jnp.float32)])
    pos32 = jnp.maximum(pos, 0).astype(jnp.int32)

    norm2d = _sc_norm(row2d, col2d, ew2d)

    m1 = _tc_matmul(x_pad, W1)
    a1 = _sc_agg(row2d, col2d, norm2d, m1)
    m2 = _tc_comb_mm(a1, b1, W2)
    a2 = _sc_agg(row2d, col2d, norm2d, m2)
    h2 = _tc_comb(a2, b2)
    g2 = _sc_agg_gather(row2d, col2d, norm2d, h2, pos32)

    emb = _tc_emb(g2, pos.reshape(ACT, 1).astype(jnp.int32), W3, b3)
    flat = emb.reshape(1, ACT * EMB)
    return _tc_mlp(flat, Wf1, bf1, Wf2, bf2, Wf3, bf3)


# speedup vs baseline: 1.0171x; 1.0171x over previous
import functools

import jax
import jax.numpy as jnp
from jax import lax
from jax.experimental import pallas as pl
from jax.experimental.pallas import tpu as pltpu
from jax.experimental.pallas import tpu_sc as plsc

N_NODES = 10000
D_FEAT = 128
HID = 16
EMB = 100
ACT = 64

NC, NS, L = 2, 16, 16
NW = NC * NS
NP = 10240
ROWS_PT = NP // NS
CHUNK = 128
N_EDGES = 320000
E_TOT = N_EDGES + N_NODES
CPT = -(-E_TOT // (NW * CHUNK))
CPT += CPT % 2
EPT = CPT * CHUNK
EPAD = EPT * NW
DCPT = CPT * NC

_mesh = plsc.VectorSubcoreMesh(core_axis_name="c", subcore_axis_name="s")
_sc_params = pltpu.CompilerParams(use_tc_tiling_on_sc=False,
                                  needs_layout_passes=False)


def _rsqrt_newton(d):
    y = plsc.bitcast(jnp.int32(0x5F3759DF) - (plsc.bitcast(d, jnp.int32) >> 1),
                     jnp.float32)
    for _ in range(3):
        y = y * (1.5 - 0.5 * d * y * y)
    return y


@functools.partial(
    pl.kernel,
    out_type=[
        jax.ShapeDtypeStruct((EPAD // CHUNK, CHUNK), jnp.float32),
        jax.ShapeDtypeStruct((NC, NP, HID), jnp.float32),
    ],
    mesh=_mesh,
    compiler_params=_sc_params,
    scratch_types=[
        pltpu.VMEM((DCPT, CHUNK), jnp.int32),
        pltpu.VMEM((DCPT, CHUNK), jnp.float32),
        pltpu.VMEM((CPT, CHUNK), jnp.int32),
        pltpu.VMEM((CPT, CHUNK), jnp.float32),
        pltpu.VMEM((NP,), jnp.float32),
        pltpu.VMEM((ROWS_PT,), jnp.float32),
        pltpu.VMEM((CHUNK, HID), jnp.float32),
        pltpu.VMEM((CHUNK, HID), jnp.float32),
        pltpu.VMEM((CHUNK, HID), jnp.float32),
        pltpu.VMEM((CHUNK, HID), jnp.float32),
        pltpu.VMEM((ROWS_PT, HID), jnp.float32),
        pltpu.VMEM_SHARED((NP,), jnp.float32),
        pltpu.VMEM_SHARED((NP,), jnp.float32),
        pltpu.VMEM_SHARED((NP, HID), jnp.float32),
        pltpu.VMEM_SHARED((NP, HID), jnp.float32),
        pltpu.SemaphoreType.DMA,
        pltpu.SemaphoreType.DMA,
        pltpu.SemaphoreType.DMA,
        pltpu.SemaphoreType.DMA,
        pltpu.SemaphoreType.DMA,
    ],
)
def _sc_norm_agg(row_hbm, col_hbm, ew_hbm, m_hbm, norm_hbm, out_hbm,
                 c_buf, w_buf, r_buf, n_buf, dis_buf, d_buf,
                 g0, g1, s0, s1, o_buf, dacc, dis_sh, m_sh, acc,
                 dsem0, dsem1, dsem2, dsem3, nsem):
    cid = lax.axis_index("c")
    sid = lax.axis_index("s")
    wid = sid * NC + cid

    def zb(i, carry):
        d_buf[pl.ds(i * L, L)] = jnp.zeros((L,), jnp.float32)
        return carry

    lax.fori_loop(0, ROWS_PT // L, zb, 0)
    pltpu.sync_copy(d_buf, dacc.at[pl.ds(sid * ROWS_PT, ROWS_PT)])
    pltpu.sync_copy(col_hbm.at[pl.ds(sid * DCPT, DCPT), :], c_buf)
    pltpu.sync_copy(ew_hbm.at[pl.ds(sid * DCPT, DCPT), :], w_buf)
    plsc.subcore_barrier()

    dsems = [dsem0, dsem1, dsem2, dsem3]
    for t in range(4):
        pltpu.async_copy(w_buf.at[t], dacc.at[c_buf.at[t]], dsems[t], add=True)

    def deg_body(i, carry):
        j = 4 * i
        for t in range(4):
            pltpu.make_async_copy(w_buf.at[j + t], dacc.at[c_buf.at[j + t]],
                                  dsems[t]).wait()

            @pl.when(j + t + 4 < DCPT)
            def _():
                pltpu.async_copy(w_buf.at[j + t + 4],
                                 dacc.at[c_buf.at[j + t + 4]],
                                 dsems[t], add=True)

        return carry

    lax.fori_loop(0, DCPT // 4, deg_body, 0)
    plsc.subcore_barrier()

    pltpu.sync_copy(dacc.at[pl.ds(sid * ROWS_PT, ROWS_PT)], d_buf)
    for i in range(ROWS_PT // L):
        sl = pl.ds(i * L, L)
        d_buf[sl] = _rsqrt_newton(d_buf[sl])
    pltpu.sync_copy(d_buf, dis_sh.at[pl.ds(sid * ROWS_PT, ROWS_PT)])
    plsc.subcore_barrier()
    pltpu.sync_copy(dis_sh, dis_buf)

    pltpu.sync_copy(row_hbm.at[pl.ds(wid * CPT, CPT), :], r_buf)
    pltpu.sync_copy(col_hbm.at[pl.ds(wid * CPT, CPT), :],
                    c_buf.at[pl.ds(0, CPT), :])
    pltpu.sync_copy(ew_hbm.at[pl.ds(wid * CPT, CPT), :], n_buf)

    def norm_body(j, carry):
        for b in range(CHUNK // L):
            sl = pl.ds(b * L, L)
            dr = plsc.load_gather(dis_buf, [r_buf[j, sl]])
            dc = plsc.load_gather(dis_buf, [c_buf[j, sl]])
            n_buf[j, sl] = dr * n_buf[j, sl] * dc
        return carry

    lax.fori_loop(0, CPT, norm_body, 0)
    pltpu.async_copy(n_buf, norm_hbm.at[pl.ds(wid * CPT, CPT), :], nsem)

    def stage(o):
        pltpu.sync_copy(m_hbm.at[pl.ds(sid * ROWS_PT, ROWS_PT), :], o)

    _agg_loop(stage, r_buf, c_buf, n_buf, g0, g1, s0, s1, o_buf, m_sh, acc,
              dsem0, dsem1, dsem2, dsem3, sid)
    pltpu.sync_copy(acc.at[pl.ds(sid * ROWS_PT, ROWS_PT), :], o_buf)
    pltpu.sync_copy(o_buf, out_hbm.at[cid, pl.ds(sid * ROWS_PT, ROWS_PT), :])
    pltpu.make_async_copy(n_buf, norm_hbm.at[pl.ds(wid * CPT, CPT), :],
                          nsem).wait()


_AGG_SCRATCH = [
    pltpu.VMEM((CPT, CHUNK), jnp.int32),
    pltpu.VMEM((CPT, CHUNK), jnp.int32),
    pltpu.VMEM((CPT, CHUNK), jnp.float32),
    pltpu.VMEM((CHUNK, HID), jnp.float32),
    pltpu.VMEM((CHUNK, HID), jnp.float32),
    pltpu.VMEM((CHUNK, HID), jnp.float32),
    pltpu.VMEM((CHUNK, HID), jnp.float32),
    pltpu.VMEM((ROWS_PT, HID), jnp.float32),
    pltpu.VMEM_SHARED((NP, HID), jnp.float32),
    pltpu.VMEM_SHARED((NP, HID), jnp.float32),
    pltpu.SemaphoreType.DMA,
    pltpu.SemaphoreType.DMA,
    pltpu.SemaphoreType.DMA,
    pltpu.SemaphoreType.DMA,
]


def _agg_main(row_hbm, col_hbm, norm_hbm, m_hbm,
              r_buf, c_buf, n_buf, g0, g1, s0, s1, o_buf, m_sh, acc,
              gsem0, gsem1, ssem0, ssem1, cid, sid, wid):
    pltpu.sync_copy(row_hbm.at[pl.ds(wid * CPT, CPT), :], r_buf)
    pltpu.sync_copy(col_hbm.at[pl.ds(wid * CPT, CPT), :], c_buf)
    pltpu.sync_copy(norm_hbm.at[pl.ds(wid * CPT, CPT), :], n_buf)

    def stage(o):
        pltpu.sync_copy(m_hbm.at[pl.ds(sid * ROWS_PT, ROWS_PT), :], o)

    _agg_loop(stage, r_buf, c_buf, n_buf, g0, g1, s0, s1, o_buf, m_sh, acc,
              gsem0, gsem1, ssem0, ssem1, sid)


def _agg_loop(stage, r_buf, c_buf, n_buf, g0, g1, s0, s1, o_buf, m_sh, acc,
              gsem0, gsem1, ssem0, ssem1, sid):
    stage(o_buf)
    pltpu.sync_copy(o_buf, m_sh.at[pl.ds(sid * ROWS_PT, ROWS_PT), :])

    def zb(i, carry):
        o_buf[i, :] = jnp.zeros((L,), jnp.float32)
        return carry

    lax.fori_loop(0, ROWS_PT, zb, 0)
    pltpu.sync_copy(o_buf, acc.at[pl.ds(sid * ROWS_PT, ROWS_PT), :])
    plsc.subcore_barrier()

    iota = lax.iota(jnp.int32, L)
    evecs = [iota + b * L for b in range(CHUNK // L)]
    fvecs = [jnp.full((L,), f, jnp.int32) for f in range(HID)]

    def scale(j, g_buf, s_buf):
        for b in range(CHUNK // L):
            nv = n_buf[j, pl.ds(b * L, L)]
            for f in range(HID):
                vals = plsc.load_gather(g_buf, [evecs[b], fvecs[f]])
                plsc.store_scatter(s_buf, [evecs[b], fvecs[f]], vals * nv)

    pltpu.async_copy(m_sh.at[r_buf.at[0]], g0, gsem0)
    pltpu.async_copy(m_sh.at[r_buf.at[1]], g1, gsem1)

    def body(i, carry):
        j = 2 * i
        pltpu.make_async_copy(m_sh.at[r_buf.at[j]], g0, gsem0).wait()

        @pl.when(i > 0)
        def _():
            pltpu.make_async_copy(s0, acc.at[c_buf.at[j]], ssem0).wait()

        scale(j, g0, s0)
        pltpu.async_copy(s0, acc.at[c_buf.at[j]], ssem0, add=True)

        @pl.when(j + 2 < CPT)
        def _():
            pltpu.async_copy(m_sh.at[r_buf.at[j + 2]], g0, gsem0)

        pltpu.make_async_copy(m_sh.at[r_buf.at[j + 1]], g1, gsem1).wait()

        @pl.when(i > 0)
        def _():
            pltpu.make_async_copy(s1, acc.at[c_buf.at[j + 1]], ssem1).wait()

        scale(j + 1, g1, s1)
        pltpu.async_copy(s1, acc.at[c_buf.at[j + 1]], ssem1, add=True)

        @pl.when(j + 3 < CPT)
        def _():
            pltpu.async_copy(m_sh.at[r_buf.at[j + 3]], g1, gsem1)

        return carry

    lax.fori_loop(0, CPT // 2, body, 0)
    pltpu.make_async_copy(s0, acc.at[c_buf.at[CPT - 2]], ssem0).wait()
    pltpu.make_async_copy(s1, acc.at[c_buf.at[CPT - 1]], ssem1).wait()
    plsc.subcore_barrier()


@functools.partial(
    pl.kernel,
    out_type=jax.ShapeDtypeStruct((NC, NP, HID), jnp.float32),
    mesh=_mesh,
    compiler_params=_sc_params,
    scratch_types=_AGG_SCRATCH,
)
def _sc_agg(row_hbm, col_hbm, norm_hbm, m_hbm, out_hbm,
            r_buf, c_buf, n_buf, g0, g1, s0, s1, o_buf, m_sh, acc,
            gsem0, gsem1, ssem0, ssem1):
    cid = lax.axis_index("c")
    sid = lax.axis_index("s")
    wid = sid * NC + cid
    _agg_main(row_hbm, col_hbm, norm_hbm, m_hbm,
              r_buf, c_buf, n_buf, g0, g1, s0, s1, o_buf, m_sh, acc,
              gsem0, gsem1, ssem0, ssem1, cid, sid, wid)
    pltpu.sync_copy(acc.at[pl.ds(sid * ROWS_PT, ROWS_PT), :], o_buf)
    pltpu.sync_copy(o_buf, out_hbm.at[cid, pl.ds(sid * ROWS_PT, ROWS_PT), :])


@functools.partial(
    pl.kernel,
    out_type=jax.ShapeDtypeStruct((NC, ACT, HID), jnp.float32),
    mesh=_mesh,
    compiler_params=_sc_params,
    scratch_types=_AGG_SCRATCH + [
        pltpu.VMEM((ACT,), jnp.int32),
        pltpu.VMEM((ACT, HID), jnp.float32),
        pltpu.VMEM((ROWS_PT, HID), jnp.float32),
        pltpu.VMEM((L,), jnp.float32),
    ],
)
def _sc_agg_gather(row_hbm, col_hbm, norm_hbm, a_hbm, b_hbm, pos_hbm, out_hbm,
                   r_buf, c_buf, n_buf, g0, g1, s0, s1, o_buf, m_sh, acc,
                   gsem0, gsem1, ssem0, ssem1, i_buf, ga_buf, h_buf, b_buf):
    cid = lax.axis_index("c")
    sid = lax.axis_index("s")
    wid = sid * NC + cid
    pltpu.sync_copy(row_hbm.at[pl.ds(wid * CPT, CPT), :], r_buf)
    pltpu.sync_copy(col_hbm.at[pl.ds(wid * CPT, CPT), :], c_buf)
    pltpu.sync_copy(norm_hbm.at[pl.ds(wid * CPT, CPT), :], n_buf)

    def stage(o):
        pltpu.sync_copy(a_hbm.at[0, pl.ds(sid * ROWS_PT, ROWS_PT), :], o)
        pltpu.sync_copy(a_hbm.at[1, pl.ds(sid * ROWS_PT, ROWS_PT), :], h_buf)
        pltpu.sync_copy(b_hbm, b_buf)
        bv = b_buf[...]

        def comb(i, carry):
            o[i, :] = jnp.maximum(o[i, :] + h_buf[i, :] + bv, 0.0)
            return carry

        lax.fori_loop(0, ROWS_PT, comb, 0)

    _agg_loop(stage, r_buf, c_buf, n_buf, g0, g1, s0, s1, o_buf, m_sh, acc,
              gsem0, gsem1, ssem0, ssem1, sid)

    @pl.when(sid == 0)
    def _():
        pltpu.sync_copy(pos_hbm, i_buf)
        pltpu.async_copy(acc.at[i_buf], ga_buf, gsem0).wait()
        pltpu.sync_copy(ga_buf, out_hbm.at[cid])


def _tc_matmul(x, W):
    def body(x_ref, w_ref, o_ref):
        o_ref[...] = jnp.dot(x_ref[...], w_ref[...],
                             preferred_element_type=jnp.float32)

    return pl.pallas_call(
        body,
        out_shape=jax.ShapeDtypeStruct((x.shape[0], W.shape[1]), jnp.float32),
    )(x, W)


def _tc_comb_mm(agg, b, W):
    def body(a_ref, b_ref, w_ref, o_ref):
        h = jnp.maximum(a_ref[0] + a_ref[1] + b_ref[...], 0.0)
        o_ref[...] = jnp.dot(h, w_ref[...], preferred_element_type=jnp.float32)

    return pl.pallas_call(
        body,
        out_shape=jax.ShapeDtypeStruct((NP, W.shape[1]), jnp.float32),
    )(agg, b.reshape(1, -1), W)


def _tc_emb(g2, pos2d, W3, b3):
    def body(g_ref, p_ref, w_ref, b_ref, o_ref):
        emb = jnp.dot(g_ref[0] + g_ref[1], w_ref[...],
                      preferred_element_type=jnp.float32) + b_ref[...]
        o_ref[...] = jnp.where(p_ref[...] == -1, jnp.float32(-1.0), emb)

    return pl.pallas_call(
        body,
        out_shape=jax.ShapeDtypeStruct((ACT, EMB), jnp.float32),
    )(g2, pos2d, W3, b3.reshape(1, -1))


def _tc_mlp(flat, Wf1, bf1, Wf2, bf2, Wf3, bf3):
    def body(f_ref, w1_ref, b1_ref, w2_ref, b2_ref, w3_ref, b3_ref, o_ref):
        z = jnp.maximum(jnp.dot(f_ref[...], w1_ref[...],
                                preferred_element_type=jnp.float32)
                        + b1_ref[...], 0.0)
        z = jnp.maximum(jnp.dot(z, w2_ref[...],
                                preferred_element_type=jnp.float32)
                        + b2_ref[...], 0.0)
        o_ref[...] = jnp.dot(z, w3_ref[...],
                             preferred_element_type=jnp.float32) + b3_ref[...]

    return pl.pallas_call(
        body,
        out_shape=jax.ShapeDtypeStruct((1, ACT), jnp.float32),
    )(flat, Wf1, bf1.reshape(1, -1), Wf2, bf2.reshape(1, -1),
      Wf3, bf3.reshape(1, -1))


def kernel(x, edge_index, edge_weight, pos, W1, b1, W2, b2, W3, b3,
           Wf1, bf1, Wf2, bf2, Wf3, bf3):
    ei = edge_index.astype(jnp.int32)
    loop = jnp.arange(N_NODES, dtype=jnp.int32)
    pad = EPAD - E_TOT
    row = jnp.concatenate([ei[0], loop, jnp.zeros((pad,), jnp.int32)])
    col = jnp.concatenate([ei[1], loop, jnp.zeros((pad,), jnp.int32)])
    ew = jnp.concatenate([edge_weight.astype(jnp.float32),
                          jnp.ones((N_NODES,), jnp.float32),
                          jnp.zeros((pad,), jnp.float32)])
    row2d = row.reshape(EPAD // CHUNK, CHUNK)
    col2d = col.reshape(EPAD // CHUNK, CHUNK)
    ew2d = ew.reshape(EPAD // CHUNK, CHUNK)

    x_pad = jnp.concatenate(
        [x.astype(jnp.float32), jnp.zeros((NP - N_NODES, D_FEAT), jnp.float32)])
    pos32 = jnp.maximum(pos, 0).astype(jnp.int32)

    m1 = _tc_matmul(x_pad, W1)
    norm2d, a1 = _sc_norm_agg(row2d, col2d, ew2d, m1)
    m2 = _tc_comb_mm(a1, b1, W2)
    a2 = _sc_agg(row2d, col2d, norm2d, m2)
    g2 = _sc_agg_gather(row2d, col2d, norm2d, a2, b2, pos32)

    emb = _tc_emb(g2, pos.reshape(ACT, 1).astype(jnp.int32), W3, b3)
    flat = emb.reshape(1, ACT * EMB)
    return _tc_mlp(flat, Wf1, bf1, Wf2, bf2, Wf3, bf3)


# speedup vs baseline: 2.1422x; 2.1061x over previous
import functools

import jax
import jax.numpy as jnp
from jax import lax
from jax.experimental import pallas as pl
from jax.experimental.pallas import tpu as pltpu
from jax.experimental.pallas import tpu_sc as plsc

N_NODES = 10000
D_FEAT = 128
HID = 16
EMB = 100
ACT = 64

NC, NS, L = 2, 16, 16
NW = NC * NS
NP = 10240
ROWS_PT = NP // NS
CHUNK = 128
N_EDGES = 320000
E_TOT = N_EDGES + N_NODES
CPT = -(-E_TOT // (NW * CHUNK))
CPT += CPT % 2
EPT = CPT * CHUNK
EPAD = EPT * NW
DCPT = CPT * NC

_mesh = plsc.VectorSubcoreMesh(core_axis_name="c", subcore_axis_name="s")
_sc_params = pltpu.CompilerParams(use_tc_tiling_on_sc=False,
                                  needs_layout_passes=False)


def _rsqrt_newton(d):
    y = plsc.bitcast(jnp.int32(0x5F3759DF) - (plsc.bitcast(d, jnp.int32) >> 1),
                     jnp.float32)
    for _ in range(3):
        y = y * (1.5 - 0.5 * d * y * y)
    return y


@functools.partial(
    pl.kernel,
    out_type=[
        jax.ShapeDtypeStruct((EPAD // CHUNK, CHUNK), jnp.float32),
        jax.ShapeDtypeStruct((NC, NP, HID), jnp.float32),
    ],
    mesh=_mesh,
    compiler_params=_sc_params,
    scratch_types=[
        pltpu.VMEM((DCPT, CHUNK), jnp.int32),
        pltpu.VMEM((DCPT, CHUNK), jnp.float32),
        pltpu.VMEM((CPT, CHUNK), jnp.int32),
        pltpu.VMEM((CPT, CHUNK), jnp.float32),
        pltpu.VMEM((NP,), jnp.float32),
        pltpu.VMEM((ROWS_PT,), jnp.float32),
        pltpu.VMEM((CHUNK, HID), jnp.float32),
        pltpu.VMEM((CHUNK, HID), jnp.float32),
        pltpu.VMEM((CHUNK, HID), jnp.float32),
        pltpu.VMEM((CHUNK, HID), jnp.float32),
        pltpu.VMEM((ROWS_PT, HID), jnp.float32),
        pltpu.VMEM_SHARED((NP,), jnp.float32),
        pltpu.VMEM_SHARED((NP,), jnp.float32),
        pltpu.VMEM_SHARED((NP, HID), jnp.float32),
        pltpu.VMEM_SHARED((NP, HID), jnp.float32),
        pltpu.SemaphoreType.DMA,
        pltpu.SemaphoreType.DMA,
        pltpu.SemaphoreType.DMA,
        pltpu.SemaphoreType.DMA,
        pltpu.SemaphoreType.DMA,
    ],
)
def _sc_norm_agg(row_hbm, col_hbm, ew_hbm, m_hbm, norm_hbm, out_hbm,
                 c_buf, w_buf, r_buf, n_buf, dis_buf, d_buf,
                 g0, g1, s0, s1, o_buf, dacc, dis_sh, m_sh, acc,
                 dsem0, dsem1, dsem2, dsem3, nsem):
    cid = lax.axis_index("c")
    sid = lax.axis_index("s")
    wid = sid * NC + cid

    def zb(i, carry):
        d_buf[pl.ds(i * L, L)] = jnp.zeros((L,), jnp.float32)
        return carry

    lax.fori_loop(0, ROWS_PT // L, zb, 0)
    pltpu.sync_copy(d_buf, dacc.at[pl.ds(sid * ROWS_PT, ROWS_PT)])
    pltpu.sync_copy(col_hbm.at[pl.ds(sid * DCPT, DCPT), :], c_buf)
    pltpu.sync_copy(ew_hbm.at[pl.ds(sid * DCPT, DCPT), :], w_buf)
    plsc.subcore_barrier()

    dsems = [dsem0, dsem1, dsem2, dsem3]
    for t in range(4):
        pltpu.async_copy(w_buf.at[t], dacc.at[c_buf.at[t]], dsems[t], add=True)

    def deg_body(i, carry):
        j = 4 * i
        for t in range(4):
            pltpu.make_async_copy(w_buf.at[j + t], dacc.at[c_buf.at[j + t]],
                                  dsems[t]).wait()

            @pl.when(j + t + 4 < DCPT)
            def _():
                pltpu.async_copy(w_buf.at[j + t + 4],
                                 dacc.at[c_buf.at[j + t + 4]],
                                 dsems[t], add=True)

        return carry

    lax.fori_loop(0, DCPT // 4, deg_body, 0)
    plsc.subcore_barrier()

    pltpu.sync_copy(dacc.at[pl.ds(sid * ROWS_PT, ROWS_PT)], d_buf)
    for i in range(ROWS_PT // L):
        sl = pl.ds(i * L, L)
        d_buf[sl] = _rsqrt_newton(d_buf[sl])
    pltpu.sync_copy(d_buf, dis_sh.at[pl.ds(sid * ROWS_PT, ROWS_PT)])
    plsc.subcore_barrier()
    pltpu.sync_copy(dis_sh, dis_buf)

    pltpu.sync_copy(row_hbm.at[pl.ds(wid * CPT, CPT), :], r_buf)
    pltpu.sync_copy(col_hbm.at[pl.ds(wid * CPT, CPT), :],
                    c_buf.at[pl.ds(0, CPT), :])
    pltpu.sync_copy(ew_hbm.at[pl.ds(wid * CPT, CPT), :], n_buf)

    def norm_body(j, carry):
        for b in range(CHUNK // L):
            sl = pl.ds(b * L, L)
            dr = plsc.load_gather(dis_buf, [r_buf[j, sl]])
            dc = plsc.load_gather(dis_buf, [c_buf[j, sl]])
            n_buf[j, sl] = dr * n_buf[j, sl] * dc
        return carry

    lax.fori_loop(0, CPT, norm_body, 0)
    pltpu.async_copy(n_buf, norm_hbm.at[pl.ds(wid * CPT, CPT), :], nsem)

    def stage(o):
        pltpu.sync_copy(m_hbm.at[pl.ds(sid * ROWS_PT, ROWS_PT), :], o)

    _agg_loop(stage, r_buf, c_buf, n_buf, g0, g1, s0, s1, o_buf, m_sh, acc,
              dsem0, dsem1, dsem2, dsem3, sid)
    pltpu.sync_copy(acc.at[pl.ds(sid * ROWS_PT, ROWS_PT), :], o_buf)
    pltpu.sync_copy(o_buf, out_hbm.at[cid, pl.ds(sid * ROWS_PT, ROWS_PT), :])
    pltpu.make_async_copy(n_buf, norm_hbm.at[pl.ds(wid * CPT, CPT), :],
                          nsem).wait()


_AGG_SCRATCH = [
    pltpu.VMEM((CPT, CHUNK), jnp.int32),
    pltpu.VMEM((CPT, CHUNK), jnp.int32),
    pltpu.VMEM((CPT, CHUNK), jnp.float32),
    pltpu.VMEM((CHUNK, HID), jnp.float32),
    pltpu.VMEM((CHUNK, HID), jnp.float32),
    pltpu.VMEM((CHUNK, HID), jnp.float32),
    pltpu.VMEM((CHUNK, HID), jnp.float32),
    pltpu.VMEM((ROWS_PT, HID), jnp.float32),
    pltpu.VMEM_SHARED((NP, HID), jnp.float32),
    pltpu.VMEM_SHARED((NP, HID), jnp.float32),
    pltpu.SemaphoreType.DMA,
    pltpu.SemaphoreType.DMA,
    pltpu.SemaphoreType.DMA,
    pltpu.SemaphoreType.DMA,
]


def _agg_main(row_hbm, col_hbm, norm_hbm, m_hbm,
              r_buf, c_buf, n_buf, g0, g1, s0, s1, o_buf, m_sh, acc,
              gsem0, gsem1, ssem0, ssem1, cid, sid, wid):
    pltpu.sync_copy(row_hbm.at[pl.ds(wid * CPT, CPT), :], r_buf)
    pltpu.sync_copy(col_hbm.at[pl.ds(wid * CPT, CPT), :], c_buf)
    pltpu.sync_copy(norm_hbm.at[pl.ds(wid * CPT, CPT), :], n_buf)

    def stage(o):
        pltpu.sync_copy(m_hbm.at[pl.ds(sid * ROWS_PT, ROWS_PT), :], o)

    _agg_loop(stage, r_buf, c_buf, n_buf, g0, g1, s0, s1, o_buf, m_sh, acc,
              gsem0, gsem1, ssem0, ssem1, sid)


def _agg_loop(stage, r_buf, c_buf, n_buf, g0, g1, s0, s1, o_buf, m_sh, acc,
              gsem0, gsem1, ssem0, ssem1, sid):
    stage(o_buf)
    pltpu.sync_copy(o_buf, m_sh.at[pl.ds(sid * ROWS_PT, ROWS_PT), :])

    def zb(i, carry):
        o_buf[i, :] = jnp.zeros((L,), jnp.float32)
        return carry

    lax.fori_loop(0, ROWS_PT, zb, 0)
    pltpu.sync_copy(o_buf, acc.at[pl.ds(sid * ROWS_PT, ROWS_PT), :])
    plsc.subcore_barrier()

    ksplat = [jnp.full((L,), k, jnp.int32) for k in range(L)]

    def scale(j, g_buf, s_buf):
        for b in range(CHUNK // L):
            nv = n_buf[j, pl.ds(b * L, L)]
            for k in range(L):
                e = b * L + k
                nsp = nv.at[ksplat[k]].get(mode="promise_in_bounds")
                s_buf[e, :] = g_buf[e, :] * nsp

    pltpu.async_copy(m_sh.at[r_buf.at[0]], g0, gsem0)
    pltpu.async_copy(m_sh.at[r_buf.at[1]], g1, gsem1)

    def body(i, carry):
        j = 2 * i
        pltpu.make_async_copy(m_sh.at[r_buf.at[j]], g0, gsem0).wait()

        @pl.when(i > 0)
        def _():
            pltpu.make_async_copy(s0, acc.at[c_buf.at[j]], ssem0).wait()

        scale(j, g0, s0)
        pltpu.async_copy(s0, acc.at[c_buf.at[j]], ssem0, add=True)

        @pl.when(j + 2 < CPT)
        def _():
            pltpu.async_copy(m_sh.at[r_buf.at[j + 2]], g0, gsem0)

        pltpu.make_async_copy(m_sh.at[r_buf.at[j + 1]], g1, gsem1).wait()

        @pl.when(i > 0)
        def _():
            pltpu.make_async_copy(s1, acc.at[c_buf.at[j + 1]], ssem1).wait()

        scale(j + 1, g1, s1)
        pltpu.async_copy(s1, acc.at[c_buf.at[j + 1]], ssem1, add=True)

        @pl.when(j + 3 < CPT)
        def _():
            pltpu.async_copy(m_sh.at[r_buf.at[j + 3]], g1, gsem1)

        return carry

    lax.fori_loop(0, CPT // 2, body, 0)
    pltpu.make_async_copy(s0, acc.at[c_buf.at[CPT - 2]], ssem0).wait()
    pltpu.make_async_copy(s1, acc.at[c_buf.at[CPT - 1]], ssem1).wait()
    plsc.subcore_barrier()


@functools.partial(
    pl.kernel,
    out_type=jax.ShapeDtypeStruct((NC, NP, HID), jnp.float32),
    mesh=_mesh,
    compiler_params=_sc_params,
    scratch_types=_AGG_SCRATCH,
)
def _sc_agg(row_hbm, col_hbm, norm_hbm, m_hbm, out_hbm,
            r_buf, c_buf, n_buf, g0, g1, s0, s1, o_buf, m_sh, acc,
            gsem0, gsem1, ssem0, ssem1):
    cid = lax.axis_index("c")
    sid = lax.axis_index("s")
    wid = sid * NC + cid
    _agg_main(row_hbm, col_hbm, norm_hbm, m_hbm,
              r_buf, c_buf, n_buf, g0, g1, s0, s1, o_buf, m_sh, acc,
              gsem0, gsem1, ssem0, ssem1, cid, sid, wid)
    pltpu.sync_copy(acc.at[pl.ds(sid * ROWS_PT, ROWS_PT), :], o_buf)
    pltpu.sync_copy(o_buf, out_hbm.at[cid, pl.ds(sid * ROWS_PT, ROWS_PT), :])


@functools.partial(
    pl.kernel,
    out_type=jax.ShapeDtypeStruct((NC, ACT, HID), jnp.float32),
    mesh=_mesh,
    compiler_params=_sc_params,
    scratch_types=_AGG_SCRATCH + [
        pltpu.VMEM((ACT,), jnp.int32),
        pltpu.VMEM((ACT, HID), jnp.float32),
        pltpu.VMEM((ROWS_PT, HID), jnp.float32),
        pltpu.VMEM((L,), jnp.float32),
    ],
)
def _sc_agg_gather(row_hbm, col_hbm, norm_hbm, a_hbm, b_hbm, pos_hbm, out_hbm,
                   r_buf, c_buf, n_buf, g0, g1, s0, s1, o_buf, m_sh, acc,
                   gsem0, gsem1, ssem0, ssem1, i_buf, ga_buf, h_buf, b_buf):
    cid = lax.axis_index("c")
    sid = lax.axis_index("s")
    wid = sid * NC + cid
    pltpu.sync_copy(row_hbm.at[pl.ds(wid * CPT, CPT), :], r_buf)
    pltpu.sync_copy(col_hbm.at[pl.ds(wid * CPT, CPT), :], c_buf)
    pltpu.sync_copy(norm_hbm.at[pl.ds(wid * CPT, CPT), :], n_buf)

    def stage(o):
        pltpu.sync_copy(a_hbm.at[0, pl.ds(sid * ROWS_PT, ROWS_PT), :], o)
        pltpu.sync_copy(a_hbm.at[1, pl.ds(sid * ROWS_PT, ROWS_PT), :], h_buf)
        pltpu.sync_copy(b_hbm, b_buf)
        bv = b_buf[...]

        def comb(i, carry):
            o[i, :] = jnp.maximum(o[i, :] + h_buf[i, :] + bv, 0.0)
            return carry

        lax.fori_loop(0, ROWS_PT, comb, 0)

    _agg_loop(stage, r_buf, c_buf, n_buf, g0, g1, s0, s1, o_buf, m_sh, acc,
              gsem0, gsem1, ssem0, ssem1, sid)

    @pl.when(sid == 0)
    def _():
        pltpu.sync_copy(pos_hbm, i_buf)
        pltpu.async_copy(acc.at[i_buf], ga_buf, gsem0).wait()
        pltpu.sync_copy(ga_buf, out_hbm.at[cid])


def _tc_matmul(x, W):
    def body(x_ref, w_ref, o_ref):
        o_ref[...] = jnp.dot(x_ref[...], w_ref[...],
                             preferred_element_type=jnp.float32)

    return pl.pallas_call(
        body,
        out_shape=jax.ShapeDtypeStruct((x.shape[0], W.shape[1]), jnp.float32),
    )(x, W)


def _tc_comb_mm(agg, b, W):
    def body(a_ref, b_ref, w_ref, o_ref):
        h = jnp.maximum(a_ref[0] + a_ref[1] + b_ref[...], 0.0)
        o_ref[...] = jnp.dot(h, w_ref[...], preferred_element_type=jnp.float32)

    return pl.pallas_call(
        body,
        out_shape=jax.ShapeDtypeStruct((NP, W.shape[1]), jnp.float32),
    )(agg, b.reshape(1, -1), W)


def _tc_emb(g2, pos2d, W3, b3):
    def body(g_ref, p_ref, w_ref, b_ref, o_ref):
        emb = jnp.dot(g_ref[0] + g_ref[1], w_ref[...],
                      preferred_element_type=jnp.float32) + b_ref[...]
        o_ref[...] = jnp.where(p_ref[...] == -1, jnp.float32(-1.0), emb)

    return pl.pallas_call(
        body,
        out_shape=jax.ShapeDtypeStruct((ACT, EMB), jnp.float32),
    )(g2, pos2d, W3, b3.reshape(1, -1))


def _tc_mlp(flat, Wf1, bf1, Wf2, bf2, Wf3, bf3):
    def body(f_ref, w1_ref, b1_ref, w2_ref, b2_ref, w3_ref, b3_ref, o_ref):
        z = jnp.maximum(jnp.dot(f_ref[...], w1_ref[...],
                                preferred_element_type=jnp.float32)
                        + b1_ref[...], 0.0)
        z = jnp.maximum(jnp.dot(z, w2_ref[...],
                                preferred_element_type=jnp.float32)
                        + b2_ref[...], 0.0)
        o_ref[...] = jnp.dot(z, w3_ref[...],
                             preferred_element_type=jnp.float32) + b3_ref[...]

    return pl.pallas_call(
        body,
        out_shape=jax.ShapeDtypeStruct((1, ACT), jnp.float32),
    )(flat, Wf1, bf1.reshape(1, -1), Wf2, bf2.reshape(1, -1),
      Wf3, bf3.reshape(1, -1))


def kernel(x, edge_index, edge_weight, pos, W1, b1, W2, b2, W3, b3,
           Wf1, bf1, Wf2, bf2, Wf3, bf3):
    ei = edge_index.astype(jnp.int32)
    loop = jnp.arange(N_NODES, dtype=jnp.int32)
    pad = EPAD - E_TOT
    row = jnp.concatenate([ei[0], loop, jnp.zeros((pad,), jnp.int32)])
    col = jnp.concatenate([ei[1], loop, jnp.zeros((pad,), jnp.int32)])
    ew = jnp.concatenate([edge_weight.astype(jnp.float32),
                          jnp.ones((N_NODES,), jnp.float32),
                          jnp.zeros((pad,), jnp.float32)])
    row2d = row.reshape(EPAD // CHUNK, CHUNK)
    col2d = col.reshape(EPAD // CHUNK, CHUNK)
    ew2d = ew.reshape(EPAD // CHUNK, CHUNK)

    x_pad = jnp.concatenate(
        [x.astype(jnp.float32), jnp.zeros((NP - N_NODES, D_FEAT), jnp.float32)])
    pos32 = jnp.maximum(pos, 0).astype(jnp.int32)

    m1 = _tc_matmul(x_pad, W1)
    norm2d, a1 = _sc_norm_agg(row2d, col2d, ew2d, m1)
    m2 = _tc_comb_mm(a1, b1, W2)
    a2 = _sc_agg(row2d, col2d, norm2d, m2)
    g2 = _sc_agg_gather(row2d, col2d, norm2d, a2, b2, pos32)

    emb = _tc_emb(g2, pos.reshape(ACT, 1).astype(jnp.int32), W3, b3)
    flat = emb.reshape(1, ACT * EMB)
    return _tc_mlp(flat, Wf1, bf1, Wf2, bf2, Wf3, bf3)


# speedup vs baseline: 2.2135x; 1.0333x over previous
import functools

import jax
import jax.numpy as jnp
from jax import lax
from jax.experimental import pallas as pl
from jax.experimental.pallas import tpu as pltpu
from jax.experimental.pallas import tpu_sc as plsc

N_NODES = 10000
D_FEAT = 128
HID = 16
EMB = 100
ACT = 64

NC, NS, L = 2, 16, 16
NW = NC * NS
NP = 10240
ROWS_PT = NP // NS
CHUNK = 128
N_EDGES = 320000
E_TOT = N_EDGES + N_NODES
CPT = -(-E_TOT // (NW * CHUNK))
CPT += CPT % 2
EPT = CPT * CHUNK
EPAD = EPT * NW
DCPT = CPT * NC

_mesh = plsc.VectorSubcoreMesh(core_axis_name="c", subcore_axis_name="s")
_sc_params = pltpu.CompilerParams(use_tc_tiling_on_sc=False,
                                  needs_layout_passes=False)


def _rsqrt_newton(d):
    y = plsc.bitcast(jnp.int32(0x5F3759DF) - (plsc.bitcast(d, jnp.int32) >> 1),
                     jnp.float32)
    for _ in range(3):
        y = y * (1.5 - 0.5 * d * y * y)
    return y


@functools.partial(
    pl.kernel,
    out_type=[
        jax.ShapeDtypeStruct((EPAD // CHUNK, CHUNK), jnp.float32),
        jax.ShapeDtypeStruct((NC, NP, HID), jnp.float32),
    ],
    mesh=_mesh,
    compiler_params=_sc_params,
    scratch_types=[
        pltpu.VMEM((DCPT, CHUNK), jnp.int32),
        pltpu.VMEM((DCPT, CHUNK), jnp.float32),
        pltpu.VMEM((CPT, CHUNK), jnp.int32),
        pltpu.VMEM((CPT, CHUNK), jnp.float32),
        pltpu.VMEM((NP,), jnp.float32),
        pltpu.VMEM((ROWS_PT,), jnp.float32),
        pltpu.VMEM((CHUNK, HID), jnp.float32),
        pltpu.VMEM((CHUNK, HID), jnp.float32),
        pltpu.VMEM((CHUNK, HID), jnp.float32),
        pltpu.VMEM((CHUNK, HID), jnp.float32),
        pltpu.VMEM((ROWS_PT, HID), jnp.float32),
        pltpu.VMEM_SHARED((NP,), jnp.float32),
        pltpu.VMEM_SHARED((NP,), jnp.float32),
        pltpu.VMEM_SHARED((NP, HID), jnp.float32),
        pltpu.VMEM_SHARED((NP, HID), jnp.float32),
        pltpu.SemaphoreType.DMA,
        pltpu.SemaphoreType.DMA,
        pltpu.SemaphoreType.DMA,
        pltpu.SemaphoreType.DMA,
        pltpu.SemaphoreType.DMA,
    ],
)
def _sc_norm_agg(row_hbm, col_hbm, ew_hbm, m_hbm, norm_hbm, out_hbm,
                 c_buf, w_buf, r_buf, n_buf, dis_buf, d_buf,
                 g0, g1, s0, s1, o_buf, dacc, dis_sh, m_sh, acc,
                 dsem0, dsem1, dsem2, dsem3, nsem):
    cid = lax.axis_index("c")
    sid = lax.axis_index("s")
    wid = sid * NC + cid

    def zb(i, carry):
        d_buf[pl.ds(i * L, L)] = jnp.zeros((L,), jnp.float32)
        return carry

    lax.fori_loop(0, ROWS_PT // L, zb, 0)
    pltpu.sync_copy(d_buf, dacc.at[pl.ds(sid * ROWS_PT, ROWS_PT)])
    pltpu.sync_copy(col_hbm.at[pl.ds(sid * DCPT, DCPT), :], c_buf)
    pltpu.sync_copy(ew_hbm.at[pl.ds(sid * DCPT, DCPT), :], w_buf)
    plsc.subcore_barrier()

    dsems = [dsem0, dsem1, dsem2, dsem3]
    for t in range(4):
        pltpu.async_copy(w_buf.at[t], dacc.at[c_buf.at[t]], dsems[t], add=True)

    def deg_body(i, carry):
        j = 4 * i
        for t in range(4):
            pltpu.make_async_copy(w_buf.at[j + t], dacc.at[c_buf.at[j + t]],
                                  dsems[t]).wait()

            @pl.when(j + t + 4 < DCPT)
            def _():
                pltpu.async_copy(w_buf.at[j + t + 4],
                                 dacc.at[c_buf.at[j + t + 4]],
                                 dsems[t], add=True)

        return carry

    lax.fori_loop(0, DCPT // 4, deg_body, 0)
    plsc.subcore_barrier()

    pltpu.sync_copy(dacc.at[pl.ds(sid * ROWS_PT, ROWS_PT)], d_buf)
    for i in range(ROWS_PT // L):
        sl = pl.ds(i * L, L)
        d_buf[sl] = _rsqrt_newton(d_buf[sl])
    pltpu.sync_copy(d_buf, dis_sh.at[pl.ds(sid * ROWS_PT, ROWS_PT)])
    plsc.subcore_barrier()
    pltpu.sync_copy(dis_sh, dis_buf)

    pltpu.sync_copy(row_hbm.at[pl.ds(wid * CPT, CPT), :], r_buf)
    pltpu.sync_copy(col_hbm.at[pl.ds(wid * CPT, CPT), :],
                    c_buf.at[pl.ds(0, CPT), :])
    pltpu.sync_copy(ew_hbm.at[pl.ds(wid * CPT, CPT), :], n_buf)

    def norm_body(j, carry):
        for b in range(CHUNK // L):
            sl = pl.ds(b * L, L)
            dr = plsc.load_gather(dis_buf, [r_buf[j, sl]])
            dc = plsc.load_gather(dis_buf, [c_buf[j, sl]])
            n_buf[j, sl] = dr * n_buf[j, sl] * dc
        return carry

    lax.fori_loop(0, CPT, norm_body, 0)
    pltpu.async_copy(n_buf, norm_hbm.at[pl.ds(wid * CPT, CPT), :], nsem)

    def stage(o):
        pltpu.sync_copy(m_hbm.at[pl.ds(sid * ROWS_PT, ROWS_PT), :], o)

    _agg_loop(stage, r_buf, c_buf, n_buf, g0, g1, s0, s1, o_buf, m_sh, acc,
              dsem0, dsem1, dsem2, dsem3, sid)
    pltpu.sync_copy(acc.at[pl.ds(sid * ROWS_PT, ROWS_PT), :], o_buf)
    pltpu.sync_copy(o_buf, out_hbm.at[cid, pl.ds(sid * ROWS_PT, ROWS_PT), :])
    pltpu.make_async_copy(n_buf, norm_hbm.at[pl.ds(wid * CPT, CPT), :],
                          nsem).wait()


_AGG_SCRATCH = [
    pltpu.VMEM((CPT, CHUNK), jnp.int32),
    pltpu.VMEM((CPT, CHUNK), jnp.int32),
    pltpu.VMEM((CPT, CHUNK), jnp.float32),
    pltpu.VMEM((CHUNK, HID), jnp.float32),
    pltpu.VMEM((CHUNK, HID), jnp.float32),
    pltpu.VMEM((CHUNK, HID), jnp.float32),
    pltpu.VMEM((CHUNK, HID), jnp.float32),
    pltpu.VMEM((ROWS_PT, HID), jnp.float32),
    pltpu.VMEM_SHARED((NP, HID), jnp.float32),
    pltpu.VMEM_SHARED((NP, HID), jnp.float32),
    pltpu.SemaphoreType.DMA,
    pltpu.SemaphoreType.DMA,
    pltpu.SemaphoreType.DMA,
    pltpu.SemaphoreType.DMA,
]


def _agg_main(row_hbm, col_hbm, norm_hbm, m_hbm,
              r_buf, c_buf, n_buf, g0, g1, s0, s1, o_buf, m_sh, acc,
              gsem0, gsem1, ssem0, ssem1, cid, sid, wid):
    pltpu.sync_copy(row_hbm.at[pl.ds(wid * CPT, CPT), :], r_buf)
    pltpu.sync_copy(col_hbm.at[pl.ds(wid * CPT, CPT), :], c_buf)
    pltpu.sync_copy(norm_hbm.at[pl.ds(wid * CPT, CPT), :], n_buf)

    def stage(o):
        pltpu.sync_copy(m_hbm.at[pl.ds(sid * ROWS_PT, ROWS_PT), :], o)

    _agg_loop(stage, r_buf, c_buf, n_buf, g0, g1, s0, s1, o_buf, m_sh, acc,
              gsem0, gsem1, ssem0, ssem1, sid)


def _agg_loop(stage, r_buf, c_buf, n_buf, g0, g1, s0, s1, o_buf, m_sh, acc,
              gsem0, gsem1, ssem0, ssem1, sid):
    stage(o_buf)
    pltpu.sync_copy(o_buf, m_sh.at[pl.ds(sid * ROWS_PT, ROWS_PT), :])

    def zb(i, carry):
        o_buf[i, :] = jnp.zeros((L,), jnp.float32)
        return carry

    lax.fori_loop(0, ROWS_PT, zb, 0)
    pltpu.sync_copy(o_buf, acc.at[pl.ds(sid * ROWS_PT, ROWS_PT), :])
    plsc.subcore_barrier()

    ksplat = [jnp.full((L,), k, jnp.int32) for k in range(L)]

    def scale(j, g_buf, s_buf):
        for b in range(CHUNK // L):
            nv = n_buf[j, pl.ds(b * L, L)]
            for k in range(L):
                e = b * L + k
                nsp = nv.at[ksplat[k]].get(mode="promise_in_bounds")
                s_buf[e, :] = g_buf[e, :] * nsp

    pltpu.async_copy(m_sh.at[r_buf.at[0]], g0, gsem0)
    pltpu.async_copy(m_sh.at[r_buf.at[1]], g1, gsem1)

    def body(i, carry):
        j = 2 * i
        pltpu.make_async_copy(m_sh.at[r_buf.at[j]], g0, gsem0).wait()

        @pl.when(i > 0)
        def _():
            pltpu.make_async_copy(s0, acc.at[c_buf.at[j]], ssem0).wait()

        scale(j, g0, s0)
        pltpu.async_copy(s0, acc.at[c_buf.at[j]], ssem0, add=True)

        @pl.when(j + 2 < CPT)
        def _():
            pltpu.async_copy(m_sh.at[r_buf.at[j + 2]], g0, gsem0)

        pltpu.make_async_copy(m_sh.at[r_buf.at[j + 1]], g1, gsem1).wait()

        @pl.when(i > 0)
        def _():
            pltpu.make_async_copy(s1, acc.at[c_buf.at[j + 1]], ssem1).wait()

        scale(j + 1, g1, s1)
        pltpu.async_copy(s1, acc.at[c_buf.at[j + 1]], ssem1, add=True)

        @pl.when(j + 3 < CPT)
        def _():
            pltpu.async_copy(m_sh.at[r_buf.at[j + 3]], g1, gsem1)

        return carry

    lax.fori_loop(0, CPT // 2, body, 0)
    pltpu.make_async_copy(s0, acc.at[c_buf.at[CPT - 2]], ssem0).wait()
    pltpu.make_async_copy(s1, acc.at[c_buf.at[CPT - 1]], ssem1).wait()
    plsc.subcore_barrier()


@functools.partial(
    pl.kernel,
    out_type=jax.ShapeDtypeStruct((NC, NP, HID), jnp.float32),
    mesh=_mesh,
    compiler_params=_sc_params,
    scratch_types=_AGG_SCRATCH,
)
def _sc_agg(row_hbm, col_hbm, norm_hbm, m_hbm, out_hbm,
            r_buf, c_buf, n_buf, g0, g1, s0, s1, o_buf, m_sh, acc,
            gsem0, gsem1, ssem0, ssem1):
    cid = lax.axis_index("c")
    sid = lax.axis_index("s")
    wid = sid * NC + cid
    _agg_main(row_hbm, col_hbm, norm_hbm, m_hbm,
              r_buf, c_buf, n_buf, g0, g1, s0, s1, o_buf, m_sh, acc,
              gsem0, gsem1, ssem0, ssem1, cid, sid, wid)
    pltpu.sync_copy(acc.at[pl.ds(sid * ROWS_PT, ROWS_PT), :], o_buf)
    pltpu.sync_copy(o_buf, out_hbm.at[cid, pl.ds(sid * ROWS_PT, ROWS_PT), :])


@functools.partial(
    pl.kernel,
    out_type=jax.ShapeDtypeStruct((NC, ACT, HID), jnp.float32),
    mesh=_mesh,
    compiler_params=_sc_params,
    scratch_types=_AGG_SCRATCH + [
        pltpu.VMEM((ACT,), jnp.int32),
        pltpu.VMEM((ACT, HID), jnp.float32),
        pltpu.VMEM((ROWS_PT, HID), jnp.float32),
        pltpu.VMEM((L,), jnp.float32),
    ],
)
def _sc_agg_gather(row_hbm, col_hbm, norm_hbm, a_hbm, b_hbm, pos_hbm, out_hbm,
                   r_buf, c_buf, n_buf, g0, g1, s0, s1, o_buf, m_sh, acc,
                   gsem0, gsem1, ssem0, ssem1, i_buf, ga_buf, h_buf, b_buf):
    cid = lax.axis_index("c")
    sid = lax.axis_index("s")
    wid = sid * NC + cid
    pltpu.sync_copy(row_hbm.at[pl.ds(wid * CPT, CPT), :], r_buf)
    pltpu.sync_copy(col_hbm.at[pl.ds(wid * CPT, CPT), :], c_buf)
    pltpu.sync_copy(norm_hbm.at[pl.ds(wid * CPT, CPT), :], n_buf)

    def stage(o):
        pltpu.sync_copy(a_hbm.at[0, pl.ds(sid * ROWS_PT, ROWS_PT), :], o)
        pltpu.sync_copy(a_hbm.at[1, pl.ds(sid * ROWS_PT, ROWS_PT), :], h_buf)
        pltpu.sync_copy(b_hbm, b_buf)
        bv = b_buf[...]

        def comb(i, carry):
            o[i, :] = jnp.maximum(o[i, :] + h_buf[i, :] + bv, 0.0)
            return carry

        lax.fori_loop(0, ROWS_PT, comb, 0)

    _agg_loop(stage, r_buf, c_buf, n_buf, g0, g1, s0, s1, o_buf, m_sh, acc,
              gsem0, gsem1, ssem0, ssem1, sid)

    @pl.when(sid == 0)
    def _():
        pltpu.sync_copy(pos_hbm, i_buf)
        pltpu.async_copy(acc.at[i_buf], ga_buf, gsem0).wait()
        pltpu.sync_copy(ga_buf, out_hbm.at[cid])


def _tc_matmul(x, W):
    def body(x_ref, w_ref, o_ref):
        o_ref[pl.ds(0, N_NODES), :] = jnp.dot(
            x_ref[...], w_ref[...], preferred_element_type=jnp.float32)
        o_ref[pl.ds(N_NODES, NP - N_NODES), :] = jnp.zeros(
            (NP - N_NODES, W.shape[1]), jnp.float32)

    return pl.pallas_call(
        body,
        out_shape=jax.ShapeDtypeStruct((NP, W.shape[1]), jnp.float32),
    )(x, W)


def _tc_comb_mm(agg, b, W):
    def body(a_ref, b_ref, w_ref, o_ref):
        h = jnp.maximum(a_ref[0] + a_ref[1] + b_ref[...], 0.0)
        o_ref[...] = jnp.dot(h, w_ref[...], preferred_element_type=jnp.float32)

    return pl.pallas_call(
        body,
        out_shape=jax.ShapeDtypeStruct((NP, W.shape[1]), jnp.float32),
    )(agg, b.reshape(1, -1), W)


def _tc_head(g2, pos2d, W3, b3, Wf1, bf1, Wf2, bf2, Wf3, bf3):
    def body(g_ref, p_ref, w_ref, b_ref, w1_ref, b1_ref, w2_ref, b2_ref,
             w3_ref, b3_ref, o_ref):
        emb = jnp.dot(g_ref[0] + g_ref[1], w_ref[...],
                      preferred_element_type=jnp.float32) + b_ref[...]
        emb = jnp.where(p_ref[...] == -1, jnp.float32(-1.0), emb)
        z = b1_ref[...]
        for i in range(ACT):
            z = z + jnp.dot(emb[i:i + 1, :], w1_ref[pl.ds(i * EMB, EMB), :],
                            preferred_element_type=jnp.float32)
        z = jnp.maximum(z, 0.0)
        z = jnp.maximum(jnp.dot(z, w2_ref[...],
                                preferred_element_type=jnp.float32)
                        + b2_ref[...], 0.0)
        o_ref[...] = jnp.dot(z, w3_ref[...],
                             preferred_element_type=jnp.float32) + b3_ref[...]

    return pl.pallas_call(
        body,
        out_shape=jax.ShapeDtypeStruct((1, ACT), jnp.float32),
    )(g2, pos2d, W3, b3.reshape(1, -1), Wf1, bf1.reshape(1, -1),
      Wf2, bf2.reshape(1, -1), Wf3, bf3.reshape(1, -1))


def kernel(x, edge_index, edge_weight, pos, W1, b1, W2, b2, W3, b3,
           Wf1, bf1, Wf2, bf2, Wf3, bf3):
    ei = edge_index.astype(jnp.int32)
    loop = jnp.arange(N_NODES, dtype=jnp.int32)
    pad = EPAD - E_TOT
    row = jnp.concatenate([ei[0], loop, jnp.zeros((pad,), jnp.int32)])
    col = jnp.concatenate([ei[1], loop, jnp.zeros((pad,), jnp.int32)])
    ew = jnp.concatenate([edge_weight.astype(jnp.float32),
                          jnp.ones((N_NODES,), jnp.float32),
                          jnp.zeros((pad,), jnp.float32)])
    row2d = row.reshape(EPAD // CHUNK, CHUNK)
    col2d = col.reshape(EPAD // CHUNK, CHUNK)
    ew2d = ew.reshape(EPAD // CHUNK, CHUNK)

    pos32 = jnp.maximum(pos, 0).astype(jnp.int32)

    m1 = _tc_matmul(x.astype(jnp.float32), W1)
    norm2d, a1 = _sc_norm_agg(row2d, col2d, ew2d, m1)
    m2 = _tc_comb_mm(a1, b1, W2)
    a2 = _sc_agg(row2d, col2d, norm2d, m2)
    g2 = _sc_agg_gather(row2d, col2d, norm2d, a2, b2, pos32)

    return _tc_head(g2, pos.reshape(ACT, 1).astype(jnp.int32), W3, b3,
                    Wf1, bf1, Wf2, bf2, Wf3, bf3)


# speedup vs baseline: 2.2744x; 1.0275x over previous
import functools

import jax
import jax.numpy as jnp
from jax import lax
from jax.experimental import pallas as pl
from jax.experimental.pallas import tpu as pltpu
from jax.experimental.pallas import tpu_sc as plsc

N_NODES = 10000
D_FEAT = 128
HID = 16
EMB = 100
ACT = 64

NC, NS, L = 2, 16, 16
NW = NC * NS
NP = 10240
ROWS_PT = NP // NS
CHUNK = 128
N_EDGES = 320000
E_TOT = N_EDGES + N_NODES
CPT = -(-E_TOT // (NW * CHUNK))
CPT += CPT % 2
EPT = CPT * CHUNK
EPAD = EPT * NW
DCPT = CPT * NC

_mesh = plsc.VectorSubcoreMesh(core_axis_name="c", subcore_axis_name="s")
_sc_params = pltpu.CompilerParams(use_tc_tiling_on_sc=False,
                                  needs_layout_passes=False)


def _rsqrt_newton(d):
    y = plsc.bitcast(jnp.int32(0x5F3759DF) - (plsc.bitcast(d, jnp.int32) >> 1),
                     jnp.float32)
    for _ in range(3):
        y = y * (1.5 - 0.5 * d * y * y)
    return y


@functools.partial(
    pl.kernel,
    out_type=[
        jax.ShapeDtypeStruct((EPAD // CHUNK, CHUNK), jnp.float32),
        jax.ShapeDtypeStruct((NC, NP, HID), jnp.float32),
    ],
    mesh=_mesh,
    compiler_params=_sc_params,
    scratch_types=[
        pltpu.VMEM((DCPT, CHUNK), jnp.int32),
        pltpu.VMEM((DCPT, CHUNK), jnp.float32),
        pltpu.VMEM((CPT, CHUNK), jnp.int32),
        pltpu.VMEM((CPT, CHUNK), jnp.float32),
        pltpu.VMEM((NP,), jnp.float32),
        pltpu.VMEM((ROWS_PT,), jnp.float32),
        pltpu.VMEM((CHUNK, HID), jnp.float32),
        pltpu.VMEM((CHUNK, HID), jnp.float32),
        pltpu.VMEM((CHUNK, HID), jnp.float32),
        pltpu.VMEM((CHUNK, HID), jnp.float32),
        pltpu.VMEM((ROWS_PT, HID), jnp.float32),
        pltpu.VMEM_SHARED((NP,), jnp.float32),
        pltpu.VMEM_SHARED((NP,), jnp.float32),
        pltpu.VMEM_SHARED((NP, HID), jnp.float32),
        pltpu.VMEM_SHARED((NP, HID), jnp.float32),
        pltpu.SemaphoreType.DMA,
        pltpu.SemaphoreType.DMA,
        pltpu.SemaphoreType.DMA,
        pltpu.SemaphoreType.DMA,
        pltpu.SemaphoreType.DMA,
    ],
)
def _sc_norm_agg(row_hbm, col_hbm, ew_hbm, m_hbm, norm_hbm, out_hbm,
                 c_buf, w_buf, r_buf, n_buf, dis_buf, d_buf,
                 g0, g1, s0, s1, o_buf, dacc, dis_sh, m_sh, acc,
                 dsem0, dsem1, dsem2, dsem3, nsem):
    cid = lax.axis_index("c")
    sid = lax.axis_index("s")
    wid = sid * NC + cid

    def zb(i, carry):
        d_buf[pl.ds(i * L, L)] = jnp.zeros((L,), jnp.float32)
        return carry

    lax.fori_loop(0, ROWS_PT // L, zb, 0)
    pltpu.sync_copy(d_buf, dacc.at[pl.ds(sid * ROWS_PT, ROWS_PT)])
    pltpu.sync_copy(col_hbm.at[pl.ds(sid * DCPT, DCPT), :], c_buf)
    pltpu.sync_copy(ew_hbm.at[pl.ds(sid * DCPT, DCPT), :], w_buf)
    plsc.subcore_barrier()

    dsems = [dsem0, dsem1, dsem2, dsem3]
    for t in range(4):
        pltpu.async_copy(w_buf.at[t], dacc.at[c_buf.at[t]], dsems[t], add=True)

    def deg_body(i, carry):
        j = 4 * i
        for t in range(4):
            pltpu.make_async_copy(w_buf.at[j + t], dacc.at[c_buf.at[j + t]],
                                  dsems[t]).wait()

            @pl.when(j + t + 4 < DCPT)
            def _():
                pltpu.async_copy(w_buf.at[j + t + 4],
                                 dacc.at[c_buf.at[j + t + 4]],
                                 dsems[t], add=True)

        return carry

    lax.fori_loop(0, DCPT // 4, deg_body, 0)
    plsc.subcore_barrier()

    pltpu.sync_copy(dacc.at[pl.ds(sid * ROWS_PT, ROWS_PT)], d_buf)
    for i in range(ROWS_PT // L):
        sl = pl.ds(i * L, L)
        d_buf[sl] = _rsqrt_newton(d_buf[sl])
    pltpu.sync_copy(d_buf, dis_sh.at[pl.ds(sid * ROWS_PT, ROWS_PT)])
    plsc.subcore_barrier()
    pltpu.sync_copy(dis_sh, dis_buf)

    pltpu.sync_copy(row_hbm.at[pl.ds(wid * CPT, CPT), :], r_buf)
    pltpu.sync_copy(col_hbm.at[pl.ds(wid * CPT, CPT), :],
                    c_buf.at[pl.ds(0, CPT), :])
    pltpu.sync_copy(ew_hbm.at[pl.ds(wid * CPT, CPT), :], n_buf)

    def norm_body(j, carry):
        for b in range(CHUNK // L):
            sl = pl.ds(b * L, L)
            dr = plsc.load_gather(dis_buf, [r_buf[j, sl]])
            dc = plsc.load_gather(dis_buf, [c_buf[j, sl]])
            n_buf[j, sl] = dr * n_buf[j, sl] * dc
        return carry

    lax.fori_loop(0, CPT, norm_body, 0)
    pltpu.async_copy(n_buf, norm_hbm.at[pl.ds(wid * CPT, CPT), :], nsem)

    def stage(o):
        pltpu.sync_copy(m_hbm.at[pl.ds(sid * ROWS_PT, ROWS_PT), :], o)

    _agg_loop(stage, r_buf, c_buf, n_buf, g0, g1, s0, s1, o_buf, m_sh, acc,
              dsem0, dsem1, dsem2, dsem3, sid)
    pltpu.sync_copy(acc.at[pl.ds(sid * ROWS_PT, ROWS_PT), :], o_buf)
    pltpu.sync_copy(o_buf, out_hbm.at[cid, pl.ds(sid * ROWS_PT, ROWS_PT), :])
    pltpu.make_async_copy(n_buf, norm_hbm.at[pl.ds(wid * CPT, CPT), :],
                          nsem).wait()


_AGG_SCRATCH = [
    pltpu.VMEM((CPT, CHUNK), jnp.int32),
    pltpu.VMEM((CPT, CHUNK), jnp.int32),
    pltpu.VMEM((CPT, CHUNK), jnp.float32),
    pltpu.VMEM((CHUNK, HID), jnp.float32),
    pltpu.VMEM((CHUNK, HID), jnp.float32),
    pltpu.VMEM((CHUNK, HID), jnp.float32),
    pltpu.VMEM((CHUNK, HID), jnp.float32),
    pltpu.VMEM((ROWS_PT, HID), jnp.float32),
    pltpu.VMEM_SHARED((NP, HID), jnp.float32),
    pltpu.VMEM_SHARED((NP, HID), jnp.float32),
    pltpu.SemaphoreType.DMA,
    pltpu.SemaphoreType.DMA,
    pltpu.SemaphoreType.DMA,
    pltpu.SemaphoreType.DMA,
]


def _agg_main(row_hbm, col_hbm, norm_hbm, m_hbm,
              r_buf, c_buf, n_buf, g0, g1, s0, s1, o_buf, m_sh, acc,
              gsem0, gsem1, ssem0, ssem1, cid, sid, wid):
    pltpu.sync_copy(row_hbm.at[pl.ds(wid * CPT, CPT), :], r_buf)
    pltpu.sync_copy(col_hbm.at[pl.ds(wid * CPT, CPT), :], c_buf)
    pltpu.sync_copy(norm_hbm.at[pl.ds(wid * CPT, CPT), :], n_buf)

    def stage(o):
        pltpu.sync_copy(m_hbm.at[pl.ds(sid * ROWS_PT, ROWS_PT), :], o)

    _agg_loop(stage, r_buf, c_buf, n_buf, g0, g1, s0, s1, o_buf, m_sh, acc,
              gsem0, gsem1, ssem0, ssem1, sid)


def _agg_loop(stage, r_buf, c_buf, n_buf, g0, g1, s0, s1, o_buf, m_sh, acc,
              gsem0, gsem1, ssem0, ssem1, sid):
    stage(o_buf)
    pltpu.sync_copy(o_buf, m_sh.at[pl.ds(sid * ROWS_PT, ROWS_PT), :])

    def zb(i, carry):
        o_buf[i, :] = jnp.zeros((L,), jnp.float32)
        return carry

    lax.fori_loop(0, ROWS_PT // L, zb, 0)
    for q in range(L):
        pltpu.sync_copy(o_buf.at[pl.ds(0, ROWS_PT // L), :],
                        acc.at[pl.ds(sid * ROWS_PT + q * (ROWS_PT // L),
                                     ROWS_PT // L), :])
    plsc.subcore_barrier()

    ksplat = [jnp.full((L,), k, jnp.int32) for k in range(L)]

    def scale(j, g_buf, s_buf):
        for b in range(CHUNK // L):
            nv = n_buf[j, pl.ds(b * L, L)]
            for k in range(L):
                e = b * L + k
                nsp = nv.at[ksplat[k]].get(mode="promise_in_bounds")
                s_buf[e, :] = g_buf[e, :] * nsp

    pltpu.async_copy(m_sh.at[r_buf.at[0]], g0, gsem0)
    pltpu.async_copy(m_sh.at[r_buf.at[1]], g1, gsem1)

    def body(i, carry):
        j = 2 * i
        pltpu.make_async_copy(m_sh.at[r_buf.at[j]], g0, gsem0).wait()

        @pl.when(i > 0)
        def _():
            pltpu.make_async_copy(s0, acc.at[c_buf.at[j]], ssem0).wait()

        scale(j, g0, s0)
        pltpu.async_copy(s0, acc.at[c_buf.at[j]], ssem0, add=True)

        @pl.when(j + 2 < CPT)
        def _():
            pltpu.async_copy(m_sh.at[r_buf.at[j + 2]], g0, gsem0)

        pltpu.make_async_copy(m_sh.at[r_buf.at[j + 1]], g1, gsem1).wait()

        @pl.when(i > 0)
        def _():
            pltpu.make_async_copy(s1, acc.at[c_buf.at[j + 1]], ssem1).wait()

        scale(j + 1, g1, s1)
        pltpu.async_copy(s1, acc.at[c_buf.at[j + 1]], ssem1, add=True)

        @pl.when(j + 3 < CPT)
        def _():
            pltpu.async_copy(m_sh.at[r_buf.at[j + 3]], g1, gsem1)

        return carry

    lax.fori_loop(0, CPT // 2, body, 0)
    pltpu.make_async_copy(s0, acc.at[c_buf.at[CPT - 2]], ssem0).wait()
    pltpu.make_async_copy(s1, acc.at[c_buf.at[CPT - 1]], ssem1).wait()
    plsc.subcore_barrier()


@functools.partial(
    pl.kernel,
    out_type=jax.ShapeDtypeStruct((NC, NP, HID), jnp.float32),
    mesh=_mesh,
    compiler_params=_sc_params,
    scratch_types=_AGG_SCRATCH,
)
def _sc_agg(row_hbm, col_hbm, norm_hbm, m_hbm, out_hbm,
            r_buf, c_buf, n_buf, g0, g1, s0, s1, o_buf, m_sh, acc,
            gsem0, gsem1, ssem0, ssem1):
    cid = lax.axis_index("c")
    sid = lax.axis_index("s")
    wid = sid * NC + cid
    _agg_main(row_hbm, col_hbm, norm_hbm, m_hbm,
              r_buf, c_buf, n_buf, g0, g1, s0, s1, o_buf, m_sh, acc,
              gsem0, gsem1, ssem0, ssem1, cid, sid, wid)
    pltpu.sync_copy(acc.at[pl.ds(sid * ROWS_PT, ROWS_PT), :], o_buf)
    pltpu.sync_copy(o_buf, out_hbm.at[cid, pl.ds(sid * ROWS_PT, ROWS_PT), :])


@functools.partial(
    pl.kernel,
    out_type=jax.ShapeDtypeStruct((NC, ACT, HID), jnp.float32),
    mesh=_mesh,
    compiler_params=_sc_params,
    scratch_types=_AGG_SCRATCH + [
        pltpu.VMEM((ACT,), jnp.int32),
        pltpu.VMEM((ACT, HID), jnp.float32),
        pltpu.VMEM((ROWS_PT, HID), jnp.float32),
        pltpu.VMEM((L,), jnp.float32),
    ],
)
def _sc_agg_gather(row_hbm, col_hbm, norm_hbm, a_hbm, b_hbm, pos_hbm, out_hbm,
                   r_buf, c_buf, n_buf, g0, g1, s0, s1, o_buf, m_sh, acc,
                   gsem0, gsem1, ssem0, ssem1, i_buf, ga_buf, h_buf, b_buf):
    cid = lax.axis_index("c")
    sid = lax.axis_index("s")
    wid = sid * NC + cid
    pltpu.sync_copy(row_hbm.at[pl.ds(wid * CPT, CPT), :], r_buf)
    pltpu.sync_copy(col_hbm.at[pl.ds(wid * CPT, CPT), :], c_buf)
    pltpu.sync_copy(norm_hbm.at[pl.ds(wid * CPT, CPT), :], n_buf)

    def stage(o):
        pltpu.sync_copy(a_hbm.at[0, pl.ds(sid * ROWS_PT, ROWS_PT), :], o)
        pltpu.sync_copy(a_hbm.at[1, pl.ds(sid * ROWS_PT, ROWS_PT), :], h_buf)
        pltpu.sync_copy(b_hbm, b_buf)
        bv = b_buf[...]

        def comb(i, carry):
            o[i, :] = jnp.maximum(o[i, :] + h_buf[i, :] + bv, 0.0)
            return carry

        lax.fori_loop(0, ROWS_PT, comb, 0)

    _agg_loop(stage, r_buf, c_buf, n_buf, g0, g1, s0, s1, o_buf, m_sh, acc,
              gsem0, gsem1, ssem0, ssem1, sid)

    @pl.when(sid == 0)
    def _():
        pltpu.sync_copy(pos_hbm, i_buf)
        pltpu.async_copy(acc.at[i_buf], ga_buf, gsem0).wait()
        pltpu.sync_copy(ga_buf, out_hbm.at[cid])


def _tc_matmul(x, W):
    def body(x_ref, w_ref, o_ref):
        o_ref[pl.ds(0, N_NODES), :] = jnp.dot(
            x_ref[...], w_ref[...], preferred_element_type=jnp.float32)
        o_ref[pl.ds(N_NODES, NP - N_NODES), :] = jnp.zeros(
            (NP - N_NODES, W.shape[1]), jnp.float32)

    return pl.pallas_call(
        body,
        out_shape=jax.ShapeDtypeStruct((NP, W.shape[1]), jnp.float32),
    )(x, W)


def _tc_comb_mm(agg, b, W):
    def body(a_ref, b_ref, w_ref, o_ref):
        h = jnp.maximum(a_ref[0] + a_ref[1] + b_ref[...], 0.0)
        o_ref[...] = jnp.dot(h, w_ref[...], preferred_element_type=jnp.float32)

    return pl.pallas_call(
        body,
        out_shape=jax.ShapeDtypeStruct((NP, W.shape[1]), jnp.float32),
    )(agg, b.reshape(1, -1), W)


def _tc_head(g2, pos2d, W3, b3, Wf1, bf1, Wf2, bf2, Wf3, bf3):
    def body(g_ref, p_ref, w_ref, b_ref, w1_ref, b1_ref, w2_ref, b2_ref,
             w3_ref, b3_ref, o_ref):
        emb = jnp.dot(g_ref[0] + g_ref[1], w_ref[...],
                      preferred_element_type=jnp.float32) + b_ref[...]
        emb = jnp.where(p_ref[...] == -1, jnp.float32(-1.0), emb)
        z = b1_ref[...]
        for i in range(ACT):
            z = z + jnp.dot(emb[i:i + 1, :], w1_ref[pl.ds(i * EMB, EMB), :],
                            preferred_element_type=jnp.float32)
        z = jnp.maximum(z, 0.0)
        z = jnp.maximum(jnp.dot(z, w2_ref[...],
                                preferred_element_type=jnp.float32)
                        + b2_ref[...], 0.0)
        o_ref[...] = jnp.dot(z, w3_ref[...],
                             preferred_element_type=jnp.float32) + b3_ref[...]

    return pl.pallas_call(
        body,
        out_shape=jax.ShapeDtypeStruct((1, ACT), jnp.float32),
    )(g2, pos2d, W3, b3.reshape(1, -1), Wf1, bf1.reshape(1, -1),
      Wf2, bf2.reshape(1, -1), Wf3, bf3.reshape(1, -1))


def kernel(x, edge_index, edge_weight, pos, W1, b1, W2, b2, W3, b3,
           Wf1, bf1, Wf2, bf2, Wf3, bf3):
    ei = edge_index.astype(jnp.int32)
    loop = jnp.arange(N_NODES, dtype=jnp.int32)
    pad = EPAD - E_TOT
    row = jnp.concatenate([ei[0], loop, jnp.zeros((pad,), jnp.int32)])
    col = jnp.concatenate([ei[1], loop, jnp.zeros((pad,), jnp.int32)])
    ew = jnp.concatenate([edge_weight.astype(jnp.float32),
                          jnp.ones((N_NODES,), jnp.float32),
                          jnp.zeros((pad,), jnp.float32)])
    row2d = row.reshape(EPAD // CHUNK, CHUNK)
    col2d = col.reshape(EPAD // CHUNK, CHUNK)
    ew2d = ew.reshape(EPAD // CHUNK, CHUNK)

    pos32 = jnp.maximum(pos, 0).astype(jnp.int32)

    m1 = _tc_matmul(x.astype(jnp.float32), W1)
    norm2d, a1 = _sc_norm_agg(row2d, col2d, ew2d, m1)
    m2 = _tc_comb_mm(a1, b1, W2)
    a2 = _sc_agg(row2d, col2d, norm2d, m2)
    g2 = _sc_agg_gather(row2d, col2d, norm2d, a2, b2, pos32)

    return _tc_head(g2, pos.reshape(ACT, 1).astype(jnp.int32), W3, b3,
                    Wf1, bf1, Wf2, bf2, Wf3, bf3)
